# initial kernel scaffold (unmeasured)
import jax
import jax.numpy as jnp
from jax import lax
from jax.experimental import pallas as pl
from jax.experimental.pallas import tpu as pltpu

N_DEV = 16
B, SQ, SKV = 2, 128, 128
D_MODEL = 512
HQ_LOCAL, DH = 4, 64
HD_LOCAL = HQ_LOCAL * DH
ROWS = B * SQ

_RS_SIZES = (128, 64, 32, 16)
_RS_OFFS = (0, 128, 192, 224)


def kernel(x, Wq, K_ext, V_ext, Wo):
    x2 = x.reshape(ROWS, D_MODEL)

    def body(x_ref, wq_hbm, k_ref, v_ref, wo_hbm, out_ref,
             wq_v, wo_v, acc, rbuf, load_sems, send_sems, recv_sems):
        me = lax.axis_index("i")

        cp_wq = pltpu.make_async_copy(
            wq_hbm.at[:, pl.ds(me * HD_LOCAL, HD_LOCAL)], wq_v,
            load_sems.at[0])
        cp_wo = pltpu.make_async_copy(
            wo_hbm.at[pl.ds(me * HD_LOCAL, HD_LOCAL), :], wo_v,
            load_sems.at[1])
        cp_wq.start()
        cp_wo.start()

        bsem = pltpu.get_barrier_semaphore()
        for d in (1, 2, 4, 8):
            pl.semaphore_signal(bsem, inc=1, device_id=(me ^ d,),
                                device_id_type=pl.DeviceIdType.MESH)
        pl.semaphore_wait(bsem, 4)

        cp_wq.wait()
        cp_wo.wait()

        wq = wq_v[...].astype(jnp.bfloat16)
        wo = wo_v[...].astype(jnp.bfloat16)

        ri = lax.broadcasted_iota(jnp.int32, (SQ, SKV), 0) // 64
        cj = lax.broadcasted_iota(jnp.int32, (SQ, SKV), 1) // 64
        keep = cj <= ri

        for b in range(B):
            xb = x_ref[pl.ds(b * SQ, SQ), :].astype(jnp.bfloat16)
            qb = jnp.dot(xb, wq, preferred_element_type=jnp.float32)
            ctxs = []
            for h in range(HQ_LOCAL):
                q = qb[:, h * DH:(h + 1) * DH].astype(jnp.bfloat16)
                k = k_ref[b, :, h, :].astype(jnp.bfloat16)
                s = lax.dot_general(
                    q, k, (((1,), (1,)), ((), ())),
                    preferred_element_type=jnp.float32) * 0.125
                s = jnp.where(keep, s, -1e9)
                m = jnp.max(s, axis=-1, keepdims=True)
                w = jnp.exp(s - m)
                w = w / jnp.sum(w, axis=-1, keepdims=True)
                v = v_ref[b, :, h, :].astype(jnp.bfloat16)
                ctxs.append(jnp.dot(w.astype(jnp.bfloat16), v,
                                    preferred_element_type=jnp.float32))
            ctx_b = jnp.concatenate(ctxs, axis=1).astype(jnp.bfloat16)
            acc[pl.ds(b * SQ, SQ), :] = jnp.dot(
                ctx_b, wo, preferred_element_type=jnp.float32)

        cur_start = jnp.int32(0)
        for r, d in enumerate((8, 4, 2, 1)):
            half = _RS_SIZES[r]
            partner = me ^ d
            upper = (me & d) != 0
            keep_start = cur_start + jnp.where(upper, half, 0)
            send_start = cur_start + jnp.where(upper, 0, half)
            rdma = pltpu.make_async_remote_copy(
                src_ref=acc.at[pl.ds(send_start, half), :],
                dst_ref=rbuf.at[pl.ds(_RS_OFFS[r], half), :],
                send_sem=send_sems.at[r],
                recv_sem=recv_sems.at[r],
                device_id=(partner,),
                device_id_type=pl.DeviceIdType.MESH,
            )
            rdma.start()
            rdma.wait()
            acc[pl.ds(keep_start, half), :] = (
                acc[pl.ds(keep_start, half), :]
                + rbuf[pl.ds(_RS_OFFS[r], half), :])
            cur_start = keep_start

        out_ref[pl.ds(cur_start, 16), :] = acc[pl.ds(cur_start, 16), :]

        for j, d in enumerate((1, 2, 4, 8)):
            blk = 16 * d
            partner = me ^ d
            own_start = (me // d) * d * 16
            rdma = pltpu.make_async_remote_copy(
                src_ref=out_ref.at[pl.ds(own_start, blk), :],
                dst_ref=out_ref.at[pl.ds(own_start, blk), :],
                send_sem=send_sems.at[4 + j],
                recv_sem=recv_sems.at[4 + j],
                device_id=(partner,),
                device_id_type=pl.DeviceIdType.MESH,
            )
            rdma.start()
            rdma.wait()

    out = pl.pallas_call(
        body,
        out_shape=jax.ShapeDtypeStruct((ROWS, D_MODEL), jnp.float32),
        in_specs=[
            pl.BlockSpec(memory_space=pltpu.VMEM),
            pl.BlockSpec(memory_space=pltpu.ANY),
            pl.BlockSpec(memory_space=pltpu.VMEM),
            pl.BlockSpec(memory_space=pltpu.VMEM),
            pl.BlockSpec(memory_space=pltpu.ANY),
        ],
        out_specs=pl.BlockSpec(memory_space=pltpu.VMEM),
        scratch_shapes=[
            pltpu.VMEM((D_MODEL, HD_LOCAL), jnp.float32),
            pltpu.VMEM((HD_LOCAL, D_MODEL), jnp.float32),
            pltpu.VMEM((ROWS, D_MODEL), jnp.float32),
            pltpu.VMEM((240, D_MODEL), jnp.float32),
            pltpu.SemaphoreType.DMA((2,)),
            pltpu.SemaphoreType.DMA((8,)),
            pltpu.SemaphoreType.DMA((8,)),
        ],
        compiler_params=pltpu.CompilerParams(collective_id=0),
    )(x2, Wq, K_ext, V_ext, Wo)
    return out.reshape(B, SQ, D_MODEL)


# baseline (device time: 49064 ns/iter reference)
import jax
import jax.numpy as jnp
from jax import lax
from jax.experimental import pallas as pl
from jax.experimental.pallas import tpu as pltpu

N_DEV = 16
B, SQ, SKV = 2, 128, 128
D_MODEL = 512
HQ_LOCAL, DH = 4, 64
HD_LOCAL = HQ_LOCAL * DH
ROWS = B * SQ

_RS_SIZES = (128, 64, 32, 16)
_RS_OFFS = (0, 128, 192, 224)


def kernel(x, Wq, K_ext, V_ext, Wo):
    x2 = x.reshape(ROWS, D_MODEL)

    def body(x_ref, wq_hbm, k_ref, v_ref, wo_hbm, out_ref,
             wq_v, wo_v, acc, rbuf, load_sems, send_sems, recv_sems):
        me = lax.axis_index("i")

        cp_wq = pltpu.make_async_copy(
            wq_hbm.at[:, pl.ds(me * HD_LOCAL, HD_LOCAL)], wq_v,
            load_sems.at[0])
        cp_wo = pltpu.make_async_copy(
            wo_hbm.at[pl.ds(me * HD_LOCAL, HD_LOCAL), :], wo_v,
            load_sems.at[1])
        cp_wq.start()
        cp_wo.start()

        bsem = pltpu.get_barrier_semaphore()
        for d in (1, 2, 4, 8):
            pl.semaphore_signal(bsem, inc=1, device_id=(me ^ d,),
                                device_id_type=pl.DeviceIdType.MESH)
        pl.semaphore_wait(bsem, 4)

        cp_wq.wait()
        cp_wo.wait()

        wq = wq_v[...].astype(jnp.bfloat16)
        wo = wo_v[...].astype(jnp.bfloat16)

        ri = lax.broadcasted_iota(jnp.int32, (SQ, SKV), 0) // 64
        cj = lax.broadcasted_iota(jnp.int32, (SQ, SKV), 1) // 64
        keep = cj <= ri

        for b in range(B):
            xb = x_ref[pl.ds(b * SQ, SQ), :].astype(jnp.bfloat16)
            qb = jnp.dot(xb, wq, preferred_element_type=jnp.float32)
            ctxs = []
            for h in range(HQ_LOCAL):
                q = qb[:, h * DH:(h + 1) * DH].astype(jnp.bfloat16)
                k = k_ref[b, :, h, :].astype(jnp.bfloat16)
                s = lax.dot_general(
                    q, k, (((1,), (1,)), ((), ())),
                    preferred_element_type=jnp.float32) * 0.125
                s = jnp.where(keep, s, -1e9)
                m = jnp.max(s, axis=-1, keepdims=True)
                w = jnp.exp(s - m)
                w = w / jnp.sum(w, axis=-1, keepdims=True)
                v = v_ref[b, :, h, :].astype(jnp.bfloat16)
                ctxs.append(jnp.dot(w.astype(jnp.bfloat16), v,
                                    preferred_element_type=jnp.float32))
            ctx_b = jnp.concatenate(ctxs, axis=1).astype(jnp.bfloat16)
            acc[pl.ds(b * SQ, SQ), :] = jnp.dot(
                ctx_b, wo, preferred_element_type=jnp.float32)

        cur_start = jnp.int32(0)
        for r, d in enumerate((8, 4, 2, 1)):
            half = _RS_SIZES[r]
            partner = me ^ d
            upper = (me & d) != 0
            keep_start = cur_start + jnp.where(upper, half, 0)
            send_start = cur_start + jnp.where(upper, 0, half)
            rdma = pltpu.make_async_remote_copy(
                src_ref=acc.at[pl.ds(send_start, half), :],
                dst_ref=rbuf.at[pl.ds(_RS_OFFS[r], half), :],
                send_sem=send_sems.at[r],
                recv_sem=recv_sems.at[r],
                device_id=(partner,),
                device_id_type=pl.DeviceIdType.MESH,
            )
            rdma.start()
            rdma.wait()
            acc[pl.ds(keep_start, half), :] = (
                acc[pl.ds(keep_start, half), :]
                + rbuf[pl.ds(_RS_OFFS[r], half), :])
            cur_start = keep_start

        out_ref[pl.ds(cur_start, 16), :] = acc[pl.ds(cur_start, 16), :]

        for j, d in enumerate((1, 2, 4, 8)):
            blk = 16 * d
            partner = me ^ d
            own_start = (me // d) * d * 16
            rdma = pltpu.make_async_remote_copy(
                src_ref=out_ref.at[pl.ds(own_start, blk), :],
                dst_ref=out_ref.at[pl.ds(own_start, blk), :],
                send_sem=send_sems.at[4 + j],
                recv_sem=recv_sems.at[4 + j],
                device_id=(partner,),
                device_id_type=pl.DeviceIdType.MESH,
            )
            rdma.start()
            rdma.wait()

    out = pl.pallas_call(
        body,
        out_shape=jax.ShapeDtypeStruct((ROWS, D_MODEL), jnp.float32),
        in_specs=[
            pl.BlockSpec(memory_space=pltpu.VMEM),
            pl.BlockSpec(memory_space=pl.ANY),
            pl.BlockSpec(memory_space=pltpu.VMEM),
            pl.BlockSpec(memory_space=pltpu.VMEM),
            pl.BlockSpec(memory_space=pl.ANY),
        ],
        out_specs=pl.BlockSpec(memory_space=pltpu.VMEM),
        scratch_shapes=[
            pltpu.VMEM((D_MODEL, HD_LOCAL), jnp.float32),
            pltpu.VMEM((HD_LOCAL, D_MODEL), jnp.float32),
            pltpu.VMEM((ROWS, D_MODEL), jnp.float32),
            pltpu.VMEM((240, D_MODEL), jnp.float32),
            pltpu.SemaphoreType.DMA((2,)),
            pltpu.SemaphoreType.DMA((8,)),
            pltpu.SemaphoreType.DMA((8,)),
        ],
        compiler_params=pltpu.CompilerParams(collective_id=0),
    )(x2, Wq, K_ext, V_ext, Wo)
    return out.reshape(B, SQ, D_MODEL)


# device time: 23227 ns/iter; 2.1124x vs baseline; 2.1124x over previous
import jax
import jax.numpy as jnp
from jax import lax
from jax.experimental import pallas as pl
from jax.experimental.pallas import tpu as pltpu

N_DEV = 16
B, SQ, SKV = 2, 128, 128
D_MODEL = 512
HQ_LOCAL, DH = 4, 64
HD_LOCAL = HQ_LOCAL * DH
ROWS = B * SQ
QROWS = ROWS // 4


def kernel(x, Wq, K_ext, V_ext, Wo):
    kT = jnp.transpose(K_ext, (0, 2, 3, 1))
    vT = jnp.transpose(V_ext, (0, 2, 3, 1))
    me_out = lax.axis_index("i")
    wq_s = lax.dynamic_slice(Wq, (0, me_out * HD_LOCAL), (D_MODEL, HD_LOCAL))
    wo_s = lax.dynamic_slice(Wo, (me_out * HD_LOCAL, 0), (HD_LOCAL, D_MODEL))

    def body(x_hbm, wq_v, k_hbm, v_hbm, wo_v, out_ref,
             xv, kv, vv, acc, sbuf, prbuf, zq, zrbuf,
             load_sems, send_sems, recv_sems):
        me = lax.axis_index("i")
        z = me // 4
        p = me % 4

        loads = [
            pltpu.make_async_copy(x_hbm, xv, load_sems.at[0]),
            pltpu.make_async_copy(k_hbm, kv, load_sems.at[1]),
            pltpu.make_async_copy(v_hbm, vv, load_sems.at[2]),
        ]
        for cp in loads:
            cp.start()

        bsem = pltpu.get_barrier_semaphore()
        for k in (1, 2, 3):
            pl.semaphore_signal(bsem, inc=1,
                                device_id=(z * 4 + (p + k) % 4,),
                                device_id_type=pl.DeviceIdType.MESH)
            pl.semaphore_signal(bsem, inc=1,
                                device_id=(((z + k) % 4) * 4 + p,),
                                device_id_type=pl.DeviceIdType.MESH)
        pl.semaphore_wait(bsem, 6)

        for cp in loads:
            cp.wait()

        wq = wq_v[...].astype(jnp.bfloat16)
        wo = wo_v[...].astype(jnp.bfloat16)

        ri = lax.broadcasted_iota(jnp.int32, (SQ, SKV), 0) // 64
        cj = lax.broadcasted_iota(jnp.int32, (SQ, SKV), 1) // 64
        keep = cj <= ri

        ph0 = []
        for k in (1, 2, 3):
            qk = (p + k) % 4
            ph0.append((qk, pltpu.make_async_remote_copy(
                src_ref=sbuf.at[pl.ds(qk * QROWS, QROWS), :],
                dst_ref=prbuf.at[k - 1],
                send_sem=send_sems.at[k - 1],
                recv_sem=recv_sems.at[k - 1],
                device_id=(z * 4 + (p + k) % 4,),
                device_id_type=pl.DeviceIdType.MESH,
            )))

        for b in range(B):
            xb = xv[b].astype(jnp.bfloat16)
            qb = jnp.dot(xb, wq, preferred_element_type=jnp.float32)
            ctxs = []
            for h in range(HQ_LOCAL):
                q = qb[:, h * DH:(h + 1) * DH].astype(jnp.bfloat16)
                kk = kv[b, h].astype(jnp.bfloat16)
                s = lax.dot_general(
                    q, kk, (((1,), (0,)), ((), ())),
                    preferred_element_type=jnp.float32) * 0.125
                s = jnp.where(keep, s, -1e9)
                m = jnp.max(s, axis=-1, keepdims=True)
                w = jnp.exp(s - m)
                w = w / jnp.sum(w, axis=-1, keepdims=True)
                vh = vv[b, h].astype(jnp.bfloat16)
                ctxs.append(lax.dot_general(
                    w.astype(jnp.bfloat16), vh, (((1,), (1,)), ((), ())),
                    preferred_element_type=jnp.float32))
            ctx_b = jnp.concatenate(ctxs, axis=1).astype(jnp.bfloat16)
            partial_b = jnp.dot(ctx_b, wo,
                                preferred_element_type=jnp.float32)
            acc[pl.ds(b * SQ, SQ), :] = partial_b
            sbuf[pl.ds(b * SQ, SQ), :] = partial_b.astype(jnp.bfloat16)
            for qk, rdma in ph0:
                cond = (qk < 2) if b == 0 else (qk >= 2)
                pl.when(cond)(rdma.start)

        for _, rdma in ph0:
            rdma.wait()
        zq[...] = (acc[pl.ds(p * QROWS, QROWS), :]
                   + prbuf[0].astype(jnp.float32)
                   + prbuf[1].astype(jnp.float32)
                   + prbuf[2].astype(jnp.float32)).astype(jnp.bfloat16)

        def exchange(phase, srcs, dst_slots, partners):
            rdmas = []
            for k in (1, 2, 3):
                rdmas.append(pltpu.make_async_remote_copy(
                    src_ref=srcs[k - 1],
                    dst_ref=dst_slots[k - 1],
                    send_sem=send_sems.at[phase * 3 + k - 1],
                    recv_sem=recv_sems.at[phase * 3 + k - 1],
                    device_id=(partners[k - 1],),
                    device_id_type=pl.DeviceIdType.MESH,
                ))
            for r in rdmas:
                r.start()
            for r in rdmas:
                r.wait()

        exchange(
            1,
            [zq for _ in (1, 2, 3)],
            [zrbuf.at[k - 1] for k in (1, 2, 3)],
            [((z + k) % 4) * 4 + p for k in (1, 2, 3)],
        )
        b_idx = p // 2
        row0 = (p % 2) * QROWS
        out_ref[pl.ds(b_idx, 1), pl.ds(row0, QROWS), :] = (
            zq[...].astype(jnp.float32)
            + zrbuf[0].astype(jnp.float32)
            + zrbuf[1].astype(jnp.float32)
            + zrbuf[2].astype(jnp.float32)).astype(jnp.bfloat16)[None]

        exchange(
            2,
            [out_ref.at[pl.ds(b_idx, 1), pl.ds(row0, QROWS), :]
             for _ in (1, 2, 3)],
            [out_ref.at[pl.ds(b_idx, 1), pl.ds(row0, QROWS), :]
             for _ in (1, 2, 3)],
            [z * 4 + (p + k) % 4 for k in (1, 2, 3)],
        )

    out = pl.pallas_call(
        body,
        out_shape=jax.ShapeDtypeStruct((B, SQ, D_MODEL), jnp.bfloat16),
        in_specs=[
            pl.BlockSpec(memory_space=pltpu.MemorySpace.HBM),
            pl.BlockSpec(memory_space=pltpu.MemorySpace.VMEM),
            pl.BlockSpec(memory_space=pltpu.MemorySpace.HBM),
            pl.BlockSpec(memory_space=pltpu.MemorySpace.HBM),
            pl.BlockSpec(memory_space=pltpu.MemorySpace.VMEM),
        ],
        out_specs=pl.BlockSpec(memory_space=pltpu.VMEM),
        scratch_shapes=[
            pltpu.VMEM((B, SQ, D_MODEL), jnp.float32),
            pltpu.VMEM((B, HQ_LOCAL, DH, SKV), jnp.float32),
            pltpu.VMEM((B, HQ_LOCAL, DH, SKV), jnp.float32),
            pltpu.VMEM((ROWS, D_MODEL), jnp.float32),
            pltpu.VMEM((ROWS, D_MODEL), jnp.bfloat16),
            pltpu.VMEM((3, QROWS, D_MODEL), jnp.bfloat16),
            pltpu.VMEM((QROWS, D_MODEL), jnp.bfloat16),
            pltpu.VMEM((3, QROWS, D_MODEL), jnp.bfloat16),
            pltpu.SemaphoreType.DMA((3,)),
            pltpu.SemaphoreType.DMA((9,)),
            pltpu.SemaphoreType.DMA((9,)),
        ],
        compiler_params=pltpu.CompilerParams(collective_id=0),
    )(x, wq_s, kT, vT, wo_s)
    return out
